# baseline (speedup 1.0000x reference)
import jax
import jax.numpy as jnp
from jax.experimental import pallas as pl
from jax.experimental.pallas import tpu as pltpu

_NLANE = 128


def _prep_kernel(xt_ref, yt_ref, pm_ref, wq_ref, wk_ref, wv_ref, g_ref, b_ref,
                 q_ref, k_ref, v_ref):
    pm = pm_ref[...]
    for b in range(xt_ref.shape[0]):
        yy = jnp.dot(pm, yt_ref[b], preferred_element_type=jnp.float32,
                     precision=jax.lax.Precision.HIGHEST)
        mu = jnp.mean(yy, axis=-1, keepdims=True)
        var = jnp.mean((yy - mu) ** 2, axis=-1, keepdims=True)
        yn = (yy - mu) / jnp.sqrt(var + 1e-5) * g_ref[...] + b_ref[...]
        q_ref[b] = jnp.dot(xt_ref[b], wq_ref[...],
                           preferred_element_type=jnp.float32)
        k_ref[b] = jnp.dot(yn, wk_ref[...], preferred_element_type=jnp.float32)
        v_ref[b] = jnp.dot(yn, wv_ref[...], preferred_element_type=jnp.float32)


def _make_attn_kernel(scale, k1, k2):
    def _attn_kernel(q_ref, k_ref, v_ref, a1_ref, a2_ref, o_ref):
        q = q_ref[0]
        k = k_ref[0]
        attn = jax.lax.dot_general(
            q, k, (((1,), (1,)), ((), ())),
            preferred_element_type=jnp.float32) * scale
        n = attn.shape[0]
        n1 = attn.shape[1]
        nc = n1 // _NLANE
        ones_j = jnp.ones((_NLANE, _NLANE), jnp.float32)

        rmax = jnp.max(attn, axis=-1, keepdims=True)
        rmin = jnp.min(attn, axis=-1, keepdims=True)
        span = rmax - rmin
        a2 = attn - rmin + span
        keys = jax.lax.bitcast_convert_type(a2, jnp.int32)
        kchunks = [keys[:, c * _NLANE:(c + 1) * _NLANE] for c in range(nc)]
        lo0 = jnp.broadcast_to(jax.lax.bitcast_convert_type(span, jnp.int32),
                               (n, _NLANE))

        lo1 = lo0
        lo2 = lo0
        wv = 3 ** 14
        for _ in range(15):
            m1a = lo1 + wv
            m1b = lo1 + 2 * wv
            m2a = lo2 + wv
            m2b = lo2 + 2 * wv
            a1a = jnp.zeros((n, _NLANE), jnp.float32)
            a1b = jnp.zeros((n, _NLANE), jnp.float32)
            a2a = jnp.zeros((n, _NLANE), jnp.float32)
            a2b = jnp.zeros((n, _NLANE), jnp.float32)
            for c in range(nc):
                kc = kchunks[c]
                a1a = a1a + jnp.where(kc >= m1a, 1.0, 0.0)
                a1b = a1b + jnp.where(kc >= m1b, 1.0, 0.0)
                a2a = a2a + jnp.where(kc >= m2a, 1.0, 0.0)
                a2b = a2b + jnp.where(kc >= m2b, 1.0, 0.0)
            c1a = jnp.dot(a1a, ones_j, preferred_element_type=jnp.float32)
            c1b = jnp.dot(a1b, ones_j, preferred_element_type=jnp.float32)
            c2a = jnp.dot(a2a, ones_j, preferred_element_type=jnp.float32)
            c2b = jnp.dot(a2b, ones_j, preferred_element_type=jnp.float32)
            lo1 = jnp.where(c1b >= float(k1), m1b,
                            jnp.where(c1a >= float(k1), m1a, lo1))
            lo2 = jnp.where(c2b >= float(k2), m2b,
                            jnp.where(c2a >= float(k2), m2a, lo2))
            wv //= 3
        t1, t2 = lo1, lo2

        e = jnp.exp(attn - rmax)
        em1_c = []
        em2_c = []
        for c in range(nc):
            kc = kchunks[c]
            ec = e[:, c * _NLANE:(c + 1) * _NLANE]
            em1_c.append(jnp.where(kc >= t1, ec, 0.0))
            em2_c.append(jnp.where(kc >= t2, ec, 0.0))
        em1 = jnp.concatenate(em1_c, axis=1)
        em2 = jnp.concatenate(em2_c, axis=1)
        ones_s = jnp.ones((n1, _NLANE), jnp.float32)
        s1 = jnp.dot(em1, ones_s, preferred_element_type=jnp.float32)
        s2 = jnp.dot(em2, ones_s, preferred_element_type=jnp.float32)
        c1 = a1_ref[0] / s1
        c2 = a2_ref[0] / s2
        w_c = [em1_c[c] * c1 + em2_c[c] * c2 for c in range(nc)]
        w = jnp.concatenate(w_c, axis=1)
        o_ref[0] = jax.lax.dot_general(
            w, v_ref[0], (((1,), (0,)), ((), ())),
            preferred_element_type=jnp.float32)
    return _attn_kernel


def _proj_kernel(o_ref, wp_ref, bp_ref, xo_ref):
    xo_ref[0] = jnp.dot(o_ref[0], wp_ref[...],
                        preferred_element_type=jnp.float32) + bp_ref[...]


def kernel(x, y, Wq, Wkv, ln_g, ln_b, Wp, bp, a1, a2):
    B, C, H, W = x.shape
    N = H * W
    NH = 8
    HD = C // NH
    HP = 128
    scale = HD ** (-0.5)
    k1 = N // 2
    k2 = N // 3

    idx = jnp.arange(H)
    pm = jnp.zeros((N, N), jnp.float32)
    for kk in (3, 5, 7):
        r = (kk - 1) // 2
        band = (jnp.abs(idx[:, None] - idx[None, :]) <= r)
        band = band.astype(jnp.float32) / kk
        pm = pm + jnp.kron(band, band)

    xt = x.reshape(B, C, N).transpose(0, 2, 1)
    yt = y.reshape(B, C, N).transpose(0, 2, 1)

    def pad_heads(w):
        wr = w.reshape(C, NH, HD)
        return jnp.pad(wr, ((0, 0), (0, 0), (0, HP - HD))).reshape(C, NH * HP)

    wq_p = pad_heads(Wq)
    wk_p = pad_heads(Wkv[:, :C])
    wv_p = pad_heads(Wkv[:, C:])
    wp_p = jnp.pad(Wp.reshape(NH, HD, C),
                   ((0, 0), (0, HP - HD), (0, 0))).reshape(NH * HP, C)

    qkv_shape = jax.ShapeDtypeStruct((B, N, NH * HP), jnp.float32)
    q_p, k_p, v_p = pl.pallas_call(
        _prep_kernel,
        out_shape=(qkv_shape, qkv_shape, qkv_shape),
    )(xt, yt, pm, wq_p, wk_p, wv_p, ln_g.reshape(1, C), ln_b.reshape(1, C))

    out_all = pl.pallas_call(
        _make_attn_kernel(scale, k1, k2),
        grid=(B, NH),
        in_specs=[
            pl.BlockSpec((1, N, HP), lambda b, h: (b, 0, h)),
            pl.BlockSpec((1, N, HP), lambda b, h: (b, 0, h)),
            pl.BlockSpec((1, N, HP), lambda b, h: (b, 0, h)),
            pl.BlockSpec(memory_space=pltpu.SMEM),
            pl.BlockSpec(memory_space=pltpu.SMEM),
        ],
        out_specs=pl.BlockSpec((1, N, HP), lambda b, h: (b, 0, h)),
        out_shape=jax.ShapeDtypeStruct((B, N, NH * HP), jnp.float32),
    )(q_p, k_p, v_p, a1, a2)

    xo = pl.pallas_call(
        _proj_kernel,
        grid=(B,),
        in_specs=[
            pl.BlockSpec((1, N, NH * HP), lambda b: (b, 0, 0)),
            pl.BlockSpec((NH * HP, C), lambda b: (0, 0)),
            pl.BlockSpec((1, C), lambda b: (0, 0)),
        ],
        out_specs=pl.BlockSpec((1, N, C), lambda b: (b, 0, 0)),
        out_shape=jax.ShapeDtypeStruct((B, N, C), jnp.float32),
    )(out_all, wp_p, bp.reshape(1, C))

    return xo.reshape(B, H, W, C).transpose(0, 3, 1, 2)


# speedup vs baseline: 1.0471x; 1.0471x over previous
import jax
import jax.numpy as jnp
from jax.experimental import pallas as pl
from jax.experimental.pallas import tpu as pltpu

_NLANE = 128


def _prep_kernel(xt_ref, yt_ref, pm_ref, wq_ref, wk_ref, wv_ref, g_ref, b_ref,
                 q_ref, k_ref, v_ref):
    pm = pm_ref[...]
    for b in range(xt_ref.shape[0]):
        yy = jnp.dot(pm, yt_ref[b], preferred_element_type=jnp.float32,
                     precision=jax.lax.Precision.HIGHEST)
        mu = jnp.mean(yy, axis=-1, keepdims=True)
        var = jnp.mean((yy - mu) ** 2, axis=-1, keepdims=True)
        yn = (yy - mu) / jnp.sqrt(var + 1e-5) * g_ref[...] + b_ref[...]
        q_ref[b] = jnp.dot(xt_ref[b], wq_ref[...],
                           preferred_element_type=jnp.float32)
        k_ref[b] = jnp.dot(yn, wk_ref[...], preferred_element_type=jnp.float32)
        v_ref[b] = jnp.dot(yn, wv_ref[...], preferred_element_type=jnp.float32)


def _make_attn_kernel(scale, k1, k2):
    def _attn_kernel(q_ref, k_ref, v_ref, a1_ref, a2_ref, o_ref):
        q = q_ref[0]
        k = k_ref[0]
        attn = jax.lax.dot_general(
            q, k, (((1,), (1,)), ((), ())),
            preferred_element_type=jnp.float32) * scale
        n = attn.shape[0]
        n1 = attn.shape[1]
        nc = n1 // _NLANE
        ones_j = jnp.ones((_NLANE, _NLANE), jnp.float32)

        rmax = jnp.max(attn, axis=-1, keepdims=True)
        rmin = jnp.min(attn, axis=-1, keepdims=True)
        span = rmax - rmin
        a2 = attn - rmin + span
        keys = jax.lax.bitcast_convert_type(a2, jnp.int32)
        kchunks = [keys[:, c * _NLANE:(c + 1) * _NLANE] for c in range(nc)]
        lo0 = jnp.broadcast_to(jax.lax.bitcast_convert_type(span, jnp.int32),
                               (n, _NLANE))

        def body(i, carry):
            lo1, lo2 = carry
            w2 = jax.lax.shift_left(jnp.int32(1), jnp.int32(22) - i)
            mid1 = lo1 + w2
            mid2 = lo2 + w2
            acc1 = jnp.zeros((n, _NLANE), jnp.float32)
            acc2 = jnp.zeros((n, _NLANE), jnp.float32)
            for c in range(nc):
                kc = kchunks[c]
                acc1 = acc1 + jnp.where(kc >= mid1, 1.0, 0.0)
                acc2 = acc2 + jnp.where(kc >= mid2, 1.0, 0.0)
            cnt1 = jnp.dot(acc1, ones_j, preferred_element_type=jnp.float32)
            cnt2 = jnp.dot(acc2, ones_j, preferred_element_type=jnp.float32)
            lo1 = jnp.where(cnt1 >= float(k1), mid1, lo1)
            lo2 = jnp.where(cnt2 >= float(k2), mid2, lo2)
            return (lo1, lo2)

        t1, t2 = jax.lax.fori_loop(0, 23, body, (lo0, lo0))

        e = jnp.exp(attn - rmax)
        em1_c = []
        em2_c = []
        for c in range(nc):
            kc = kchunks[c]
            ec = e[:, c * _NLANE:(c + 1) * _NLANE]
            em1_c.append(jnp.where(kc >= t1, ec, 0.0))
            em2_c.append(jnp.where(kc >= t2, ec, 0.0))
        em1 = jnp.concatenate(em1_c, axis=1)
        em2 = jnp.concatenate(em2_c, axis=1)
        ones_s = jnp.ones((n1, _NLANE), jnp.float32)
        s1 = jnp.dot(em1, ones_s, preferred_element_type=jnp.float32)
        s2 = jnp.dot(em2, ones_s, preferred_element_type=jnp.float32)
        c1 = a1_ref[0] / s1
        c2 = a2_ref[0] / s2
        w_c = [em1_c[c] * c1 + em2_c[c] * c2 for c in range(nc)]
        w = jnp.concatenate(w_c, axis=1)
        o_ref[0] = jax.lax.dot_general(
            w, v_ref[0], (((1,), (0,)), ((), ())),
            preferred_element_type=jnp.float32)
    return _attn_kernel


def _proj_kernel(o_ref, wp_ref, bp_ref, xo_ref):
    xo_ref[0] = jnp.dot(o_ref[0], wp_ref[...],
                        preferred_element_type=jnp.float32) + bp_ref[...]


def kernel(x, y, Wq, Wkv, ln_g, ln_b, Wp, bp, a1, a2):
    B, C, H, W = x.shape
    N = H * W
    NH = 8
    HD = C // NH
    HP = 128
    scale = HD ** (-0.5)
    k1 = N // 2
    k2 = N // 3

    idx = jnp.arange(H)
    pm = jnp.zeros((N, N), jnp.float32)
    for kk in (3, 5, 7):
        r = (kk - 1) // 2
        band = (jnp.abs(idx[:, None] - idx[None, :]) <= r)
        band = band.astype(jnp.float32) / kk
        pm = pm + jnp.kron(band, band)

    xt = x.reshape(B, C, N).transpose(0, 2, 1)
    yt = y.reshape(B, C, N).transpose(0, 2, 1)

    def pad_heads(w):
        wr = w.reshape(C, NH, HD)
        return jnp.pad(wr, ((0, 0), (0, 0), (0, HP - HD))).reshape(C, NH * HP)

    wq_p = pad_heads(Wq)
    wk_p = pad_heads(Wkv[:, :C])
    wv_p = pad_heads(Wkv[:, C:])
    wp_p = jnp.pad(Wp.reshape(NH, HD, C),
                   ((0, 0), (0, HP - HD), (0, 0))).reshape(NH * HP, C)

    qkv_shape = jax.ShapeDtypeStruct((B, N, NH * HP), jnp.float32)
    q_p, k_p, v_p = pl.pallas_call(
        _prep_kernel,
        out_shape=(qkv_shape, qkv_shape, qkv_shape),
    )(xt, yt, pm, wq_p, wk_p, wv_p, ln_g.reshape(1, C), ln_b.reshape(1, C))

    out_all = pl.pallas_call(
        _make_attn_kernel(scale, k1, k2),
        grid=(B, NH),
        in_specs=[
            pl.BlockSpec((1, N, HP), lambda b, h: (b, 0, h)),
            pl.BlockSpec((1, N, HP), lambda b, h: (b, 0, h)),
            pl.BlockSpec((1, N, HP), lambda b, h: (b, 0, h)),
            pl.BlockSpec(memory_space=pltpu.SMEM),
            pl.BlockSpec(memory_space=pltpu.SMEM),
        ],
        out_specs=pl.BlockSpec((1, N, HP), lambda b, h: (b, 0, h)),
        out_shape=jax.ShapeDtypeStruct((B, N, NH * HP), jnp.float32),
    )(q_p, k_p, v_p, a1, a2)

    xo = pl.pallas_call(
        _proj_kernel,
        grid=(B,),
        in_specs=[
            pl.BlockSpec((1, N, NH * HP), lambda b: (b, 0, 0)),
            pl.BlockSpec((NH * HP, C), lambda b: (0, 0)),
            pl.BlockSpec((1, C), lambda b: (0, 0)),
        ],
        out_specs=pl.BlockSpec((1, N, C), lambda b: (b, 0, 0)),
        out_shape=jax.ShapeDtypeStruct((B, N, C), jnp.float32),
    )(out_all, wp_p, bp.reshape(1, C))

    return xo.reshape(B, H, W, C).transpose(0, 3, 1, 2)
